# baseline (speedup 1.0000x reference)
import functools

import jax
import jax.numpy as jnp
from jax import lax
from jax.experimental import pallas as pl
from jax.experimental.pallas import tpu as pltpu
from jax.experimental.pallas import tpu_sc as plsc

N = 327680
D_MODEL = 256
POS_MAX = 128
D = D_MODEL // 4

NC, NS = 2, 16
NW = NC * NS
OUT_PER_W = N // NW
B_PER_W = OUT_PER_W * 4

CHUNK = 128
NCHUNK = B_PER_W // CHUNK
NBUF = 2
GPB = 4
ROWS_PER_BUF = GPB * CHUNK
OUT_PER_BUF = ROWS_PER_BUF // 4
ROUNDS = NCHUNK // (NBUF * GPB)


def _body(pe_hbm, idx_hbm, out_hbm, pe_sh, idx_v, rows_v, *sems):
    gsem = sems[:NBUF]
    osem = sems[NBUF:]
    sid = lax.axis_index("s")
    wid = sid * NC + lax.axis_index("c")

    @pl.when(sid == 0)
    def _stage_table():
        tbl_stage = rows_v.at[0].at[pl.ds(0, POS_MAX)]
        pltpu.sync_copy(pe_hbm, tbl_stage)
        pltpu.sync_copy(tbl_stage, pe_sh)

    row0 = wid * OUT_PER_W
    pltpu.sync_copy(idx_hbm.at[wid], idx_v)
    idx_c = idx_v
    plsc.subcore_barrier()

    def fire(r, b):
        for g in range(GPB):
            j = (r * NBUF + b) * GPB + g
            pltpu.async_copy(
                pe_sh.at[idx_c.at[j]],
                rows_v.at[b].at[pl.ds(g * CHUNK, CHUNK)],
                gsem[b],
            )

    def drain(r, b):
        for g in range(GPB):
            j = (r * NBUF + b) * GPB + g
            pltpu.make_async_copy(
                pe_sh.at[idx_c.at[j]],
                rows_v.at[b].at[pl.ds(g * CHUNK, CHUNK)],
                gsem[b],
            ).wait()

    def out_slice(r, b):
        start = 4 * row0 + (r * NBUF + b) * ROWS_PER_BUF
        return out_hbm.at[pl.ds(start, ROWS_PER_BUF)]

    def buf_as_out(b):
        return rows_v.at[b]

    def scatter_start(r, b):
        pltpu.async_copy(buf_as_out(b), out_slice(r, b), osem[b])

    def scatter_wait(r, b):
        pltpu.make_async_copy(buf_as_out(b), out_slice(r, b), osem[b]).wait()

    for b in range(NBUF):
        fire(0, b)
        drain(0, b)
        scatter_start(0, b)

    def round_body(r, _):
        for b in range(NBUF):
            scatter_wait(r - 1, b)
            fire(r, b)
            drain(r, b)
            scatter_start(r, b)
        return _

    lax.fori_loop(1, ROUNDS, round_body, 0, unroll=False)

    for b in range(NBUF):
        scatter_wait(ROUNDS - 1, b)


@jax.jit
def _gather_all(pe, idx):
    mesh = plsc.VectorSubcoreMesh(
        core_axis_name="c", subcore_axis_name="s", num_cores=NC, num_subcores=NS
    )
    scratch = [
        pltpu.VMEM_SHARED((POS_MAX, D), jnp.float32),
        pltpu.VMEM((NCHUNK, CHUNK), jnp.int32),
        pltpu.VMEM((NBUF, ROWS_PER_BUF, D), jnp.float32),
    ] + [pltpu.SemaphoreType.DMA] * (2 * NBUF)
    out = pl.kernel(
        _body,
        out_type=jax.ShapeDtypeStruct((N * 4, D), jnp.float32),
        mesh=mesh,
        scratch_types=scratch,
        compiler_params=pltpu.CompilerParams(use_tc_tiling_on_sc=False),
    )(pe, idx)
    return out.reshape(N, D_MODEL)


def kernel(pos_enc, pe):
    idx = pos_enc.astype(jnp.int32).reshape(NW, NCHUNK, CHUNK)
    return _gather_all(pe, idx)


# speedup vs baseline: 1.2528x; 1.2528x over previous
import functools

import jax
import jax.numpy as jnp
from jax import lax
from jax.experimental import pallas as pl
from jax.experimental.pallas import tpu as pltpu
from jax.experimental.pallas import tpu_sc as plsc

N = 327680
D_MODEL = 256
POS_MAX = 128
D = D_MODEL // 4

NC, NS = 2, 16
NW = NC * NS

NRUN = N * 2
RUN_PER_W = NRUN // NW
CHUNK = 128
NCHUNK = RUN_PER_W // CHUNK

NBUF = 2
CPB = 2
RUNS_PER_BUF = CPB * CHUNK
ROUNDS = NCHUNK // (NBUF * CPB)


def _body(pe_hbm, ab_hbm, out_hbm, pe_sh, tbl_v, idx_v, rows_v, *sems):
    gsem = sems[:NBUF]
    osem = sems[NBUF:]
    sid = lax.axis_index("s")
    wid = sid * NC + lax.axis_index("c")

    @pl.when(sid == 0)
    def _stage_table():
        pltpu.sync_copy(pe_hbm, tbl_v)
        pltpu.sync_copy(tbl_v, pe_sh)

    run0 = wid * RUN_PER_W
    pltpu.sync_copy(ab_hbm.at[0].at[wid], idx_v.at[0])
    pltpu.sync_copy(ab_hbm.at[1].at[wid], idx_v.at[1])
    plsc.subcore_barrier()

    def fire(r, b):
        for g in range(CPB):
            j = (r * NBUF + b) * CPB + g
            for h in range(2):
                pltpu.async_copy(
                    pe_sh.at[idx_v.at[h].at[j]],
                    rows_v.at[h].at[b].at[pl.ds(g * CHUNK, CHUNK)],
                    gsem[b],
                )

    def drain(r, b):
        for g in range(CPB):
            j = (r * NBUF + b) * CPB + g
            for h in range(2):
                pltpu.make_async_copy(
                    pe_sh.at[idx_v.at[h].at[j]],
                    rows_v.at[h].at[b].at[pl.ds(g * CHUNK, CHUNK)],
                    gsem[b],
                ).wait()

    def out_slice(r, b, h):
        start = run0 + (r * NBUF + b) * RUNS_PER_BUF
        return out_hbm.at[pl.ds(start, RUNS_PER_BUF), pl.ds(h * D, D)]

    def scatter_start(r, b):
        for h in range(2):
            pltpu.async_copy(rows_v.at[h].at[b], out_slice(r, b, h), osem[b])

    def scatter_wait(r, b):
        for h in range(2):
            pltpu.make_async_copy(
                rows_v.at[h].at[b], out_slice(r, b, h), osem[b]
            ).wait()

    for b in range(NBUF):
        fire(0, b)
        drain(0, b)
        scatter_start(0, b)

    def round_body(r, _):
        for b in range(NBUF):
            scatter_wait(r - 1, b)
            fire(r, b)
            drain(r, b)
            scatter_start(r, b)
        return _

    lax.fori_loop(1, ROUNDS, round_body, 0, unroll=False)

    for b in range(NBUF):
        scatter_wait(ROUNDS - 1, b)


@jax.jit
def _gather_all(pe, ab):
    mesh = plsc.VectorSubcoreMesh(
        core_axis_name="c", subcore_axis_name="s", num_cores=NC, num_subcores=NS
    )
    scratch = [
        pltpu.VMEM_SHARED((POS_MAX, D), jnp.float32),
        pltpu.VMEM((POS_MAX, D), jnp.float32),
        pltpu.VMEM((2, NCHUNK, CHUNK), jnp.int32),
        pltpu.VMEM((2, NBUF, RUNS_PER_BUF, D), jnp.float32),
    ] + [pltpu.SemaphoreType.DMA] * (2 * NBUF)
    return pl.kernel(
        _body,
        out_type=jax.ShapeDtypeStruct((NRUN, 2 * D), jnp.float32),
        mesh=mesh,
        scratch_types=scratch,
        compiler_params=pltpu.CompilerParams(use_tc_tiling_on_sc=False),
    )(pe, ab)


def kernel(pos_enc, pe):
    idx = pos_enc.astype(jnp.int32)
    ab = idx.reshape(N // 8, 8, 2, 2).transpose(3, 0, 2, 1)
    ab = ab.reshape(2, NW, NCHUNK, CHUNK)
    res = _gather_all(pe, ab)
    return res.reshape(N // 8, 2, 8, 2 * D).transpose(0, 2, 1, 3).reshape(N, D_MODEL)
